# baseline (speedup 1.0000x reference)
import functools

import jax
import jax.numpy as jnp
from jax import lax
from jax.experimental import pallas as pl
from jax.experimental.pallas import tpu as pltpu
from jax.experimental.pallas import tpu_sc as plsc

SIM_THRESHOLD = 0.9
EPS = 1e-08

S = 9
D = 64
L = 16
NC = 2
NS = 16
NW = NC * NS
WORDS = S * D
GROUP_WORDS = L * WORDS
PAD = 17
TWORDS = WORDS * PAD
CROWS = S * S + S
MROWS = L * S

_PAIRS = [(i, j) for i in range(S) for j in range(i, S)]
_PIDX = {(i, j): k for k, (i, j) in enumerate(_PAIRS)}


def _pair(i, j):
    return _PIDX[(i, j)] if i <= j else _PIDX[(j, i)]


def _tree_or(vals):
    vals = list(vals)
    while len(vals) > 1:
        nxt = [vals[k] | vals[k + 1] for k in range(0, len(vals) - 1, 2)]
        if len(vals) % 2:
            nxt.append(vals[-1])
        vals = nxt
    return vals[0]


def _tree_sum(vals):
    vals = list(vals)
    while len(vals) > 1:
        nxt = [vals[k] + vals[k + 1] for k in range(0, len(vals) - 1, 2)]
        if len(vals) % 2:
            nxt.append(vals[-1])
        vals = nxt
    return vals[0]


def _sc_body(slots_hbm, final_hbm, mask_hbm,
             in0, in1, ob0, ob1, xT, outT, cbuf, mk0, mk1,
             isem0, isem1, osem0, osem1, msem0, msem1):
    wid = lax.axis_index("c") * NS + lax.axis_index("s")
    lane = lax.iota(jnp.int32, L)
    lane9 = lane * S
    lane17 = lane * PAD
    zero = jnp.zeros((L,), jnp.float32)
    one = jnp.ones((L,), jnp.float32)

    def in_slice(g):
        return slots_hbm.at[pl.ds((wid * 8 + g) * GROUP_WORDS, GROUP_WORDS)]

    def out_slice(g):
        return final_hbm.at[pl.ds((wid * 8 + g) * GROUP_WORDS, GROUP_WORDS)]

    def mask_slice(g):
        return mask_hbm.at[pl.ds((wid * 8 + g) * MROWS, MROWS)]

    for j in range(S):
        cbuf[pl.ds((S * S + j) * L, L)] = zero

    def compute_group(g, h, in_buf, out_buf, mkbuf, osem, msem):
        @plsc.parallel_loop(0, L)
        def tin_body(l):
            lbase = l * WORDS
            idx0 = lane17 + l
            for rc in range(WORDS // L):
                v = in_buf[pl.ds(lbase + rc * L, L)]
                plsc.store_scatter(xT, [idx0 + rc * L * PAD], v)

        def gram_pass(pairs, rows):
            def body(d, accs):
                d17 = d * PAD
                xs = {j: xT[pl.ds(j * (D * PAD) + d17, L)] for j in rows}
                return tuple(accs[k] + xs[i] * xs[j]
                             for k, (i, j) in enumerate(pairs))
            return lax.fori_loop(0, D, body, tuple(zero for _ in pairs))

        pairs_a = [(i, j) for (i, j) in _PAIRS if i < 3]
        pairs_b = [(i, j) for (i, j) in _PAIRS if i >= 3]
        G_a = gram_pass(pairs_a, range(S))
        G_b = gram_pass(pairs_b, range(3, S))
        gmap = dict(zip(pairs_a, G_a))
        gmap.update(zip(pairs_b, G_b))
        G = tuple(gmap[p] for p in _PAIRS)

        diag = [G[_pair(i, i)] for i in range(S)]
        inv_t = 1.0 / SIM_THRESHOLD
        mb = {}
        mf = {}
        for k, (i, j) in enumerate(_PAIRS):
            lhs = G[k] * inv_t - EPS
            cond = (lhs > 0.0) & (lhs * lhs > diag[i] * diag[j])
            mb[(i, j)] = cond
            mb[(j, i)] = cond
            v = jnp.where(cond, one, zero)
            mf[(i, j)] = v
            mf[(j, i)] = v

        nums = [_tree_sum(mf[(i, j)] for j in range(S)) for i in range(S)]
        multi = [nums[i] > 1.0 for i in range(S)]
        any_multi = jnp.any(_tree_or(multi))

        @pl.when(h > 0)
        def _():
            pltpu.make_async_copy(mkbuf, mask_slice(g), msem).wait()
            pltpu.make_async_copy(out_buf, out_slice(g), osem).wait()

        @pl.when(jnp.logical_not(any_multi))
        def _():
            for r in range(S):
                mkbuf[pl.ds(r * L, L)] = one

            @plsc.parallel_loop(0, L)
            def copy_body(l):
                lbase = l * WORDS
                for rc in range(WORDS // L):
                    out_buf[pl.ds(lbase + rc * L, L)] = \
                        in_buf[pl.ds(lbase + rc * L, L)]

            pltpu.async_copy(out_buf, out_slice(g), osem)

        @pl.when(any_multi)
        def _():
            inv_num = [one / (nums[i] + EPS) for i in range(S)]

            for s in range(S):
                for j in range(S):
                    c = jnp.where(multi[s], mf[(s, j)] * inv_num[s],
                                  one if j == s else zero)
                    cbuf[pl.ds((s * S + j) * L, L)] = c

            min_idx = []
            for i in range(S):
                mi = jnp.zeros((L,), jnp.int32)
                has = jnp.zeros((L,), jnp.bool_)
                for j in range(S):
                    mi = jnp.where(mb[(i, j)] & (~has),
                                   jnp.full((L,), j, jnp.int32), mi)
                    has = has | mb[(i, j)]
                min_idx.append(mi)

            for j in range(S):
                kill = jnp.zeros((L,), jnp.bool_)
                for i in range(S):
                    kill = kill | (mb[(i, j)] & multi[i] & (min_idx[i] != j))
                sm = jnp.where((nums[j] > 0.0) & (~kill), one, zero)
                plsc.store_scatter(mkbuf, [lane9 + j], sm)

            tgt = [jnp.where(multi[s], min_idx[s],
                             jnp.full((L,), s, jnp.int32))
                   for s in range(S)]

            for t0 in range(0, S, 3):
                Cs = []
                for t in range(t0, t0 + 3):
                    w = jnp.full((L,), S, jnp.int32)
                    for s in range(S):
                        w = jnp.where(tgt[s] == t,
                                      jnp.full((L,), s, jnp.int32), w)
                    wrow = w * (S * L)
                    Cs.append([plsc.load_gather(cbuf,
                                                [wrow + (j * L) + lane])
                               for j in range(S)])

                @plsc.parallel_loop(0, D)
                def out_body(d, t0=t0, Cs=Cs):
                    d17 = d * PAD
                    xs = [xT[pl.ds(j * (D * PAD) + d17, L)]
                          for j in range(S)]
                    for k in range(3):
                        t = t0 + k
                        acc = _tree_sum(Cs[k][j] * xs[j] for j in range(S))
                        outT[pl.ds(t * (D * PAD) + d17, L)] = acc

            @plsc.parallel_loop(0, L)
            def tout_body(l):
                lbase = l * WORDS
                idx0 = lane17 + l
                for rc in range(WORDS // L):
                    v = plsc.load_gather(outT, [idx0 + rc * L * PAD])
                    out_buf[pl.ds(lbase + rc * L, L)] = v

            pltpu.async_copy(out_buf, out_slice(g), osem)

        pltpu.async_copy(mkbuf, mask_slice(g), msem)

    pltpu.async_copy(in_slice(0), in0, isem0)

    def pair_body(h, carry):
        g_a = h * 2
        g_b = g_a + 1
        pltpu.async_copy(in_slice(g_b), in1, isem1)
        pltpu.make_async_copy(in_slice(g_a), in0, isem0).wait()
        compute_group(g_a, h, in0, ob0, mk0, osem0, msem0)
        g_n = jnp.minimum(g_a + 2, 7)
        pltpu.async_copy(in_slice(g_n), in0, isem0)
        pltpu.make_async_copy(in_slice(g_b), in1, isem1).wait()
        compute_group(g_b, h, in1, ob1, mk1, osem1, msem1)
        return carry

    lax.fori_loop(0, 4, pair_body, 0)

    pltpu.make_async_copy(in_slice(7), in0, isem0).wait()
    pltpu.make_async_copy(ob0, out_slice(6), osem0).wait()
    pltpu.make_async_copy(ob1, out_slice(7), osem1).wait()
    pltpu.make_async_copy(mk0, mask_slice(6), msem0).wait()
    pltpu.make_async_copy(mk1, mask_slice(7), msem1).wait()


@jax.jit
def _merged(slots_flat):
    B_total = slots_flat.shape[0] // WORDS
    run = pl.kernel(
        _sc_body,
        out_type=(
            jax.ShapeDtypeStruct((B_total * WORDS,), jnp.float32),
            jax.ShapeDtypeStruct((B_total * S,), jnp.float32),
        ),
        mesh=plsc.VectorSubcoreMesh(core_axis_name="c",
                                    subcore_axis_name="s"),
        compiler_params=pltpu.CompilerParams(needs_layout_passes=False),
        scratch_types=[
            pltpu.VMEM((GROUP_WORDS,), jnp.float32),
            pltpu.VMEM((GROUP_WORDS,), jnp.float32),
            pltpu.VMEM((GROUP_WORDS,), jnp.float32),
            pltpu.VMEM((GROUP_WORDS,), jnp.float32),
            pltpu.VMEM((TWORDS,), jnp.float32),
            pltpu.VMEM((TWORDS,), jnp.float32),
            pltpu.VMEM((CROWS * L,), jnp.float32),
            pltpu.VMEM((MROWS,), jnp.float32),
            pltpu.VMEM((MROWS,), jnp.float32),
            pltpu.SemaphoreType.DMA,
            pltpu.SemaphoreType.DMA,
            pltpu.SemaphoreType.DMA,
            pltpu.SemaphoreType.DMA,
            pltpu.SemaphoreType.DMA,
            pltpu.SemaphoreType.DMA,
        ],
    )
    return run(slots_flat)


def kernel(slots):
    B, s, d = slots.shape
    final_flat, mask_flat = _merged(slots.reshape(-1))
    return final_flat.reshape(B, s, d), mask_flat.reshape(B, s)


# speedup vs baseline: 1.0131x; 1.0131x over previous
import functools

import jax
import jax.numpy as jnp
from jax import lax
from jax.experimental import pallas as pl
from jax.experimental.pallas import tpu as pltpu
from jax.experimental.pallas import tpu_sc as plsc

SIM_THRESHOLD = 0.9
EPS = 1e-08

S = 9
D = 64
L = 16
NC = 2
NS = 16
NW = NC * NS
WORDS = S * D
GROUP_WORDS = L * WORDS
PAD = 17
TWORDS = WORDS * PAD
CROWS = S * S + S
MROWS = L * S

_PAIRS = [(i, j) for i in range(S) for j in range(i, S)]
_PIDX = {(i, j): k for k, (i, j) in enumerate(_PAIRS)}


def _pair(i, j):
    return _PIDX[(i, j)] if i <= j else _PIDX[(j, i)]


def _tree_or(vals):
    vals = list(vals)
    while len(vals) > 1:
        nxt = [vals[k] | vals[k + 1] for k in range(0, len(vals) - 1, 2)]
        if len(vals) % 2:
            nxt.append(vals[-1])
        vals = nxt
    return vals[0]


def _tree_sum(vals):
    vals = list(vals)
    while len(vals) > 1:
        nxt = [vals[k] + vals[k + 1] for k in range(0, len(vals) - 1, 2)]
        if len(vals) % 2:
            nxt.append(vals[-1])
        vals = nxt
    return vals[0]


def _sc_body(slots_hbm, final_hbm, mask_hbm,
             in0, in1, ob0, ob1, xT, outT, cbuf, mk0, mk1,
             isem0, isem1, osem0, osem1, msem0, msem1):
    wid = lax.axis_index("c") * NS + lax.axis_index("s")
    lane = lax.iota(jnp.int32, L)
    lane9 = lane * S
    lane17 = lane * PAD
    zero = jnp.zeros((L,), jnp.float32)
    one = jnp.ones((L,), jnp.float32)

    def in_slice(g):
        return slots_hbm.at[pl.ds((wid * 8 + g) * GROUP_WORDS, GROUP_WORDS)]

    def out_slice(g):
        return final_hbm.at[pl.ds((wid * 8 + g) * GROUP_WORDS, GROUP_WORDS)]

    def mask_slice(g):
        return mask_hbm.at[pl.ds((wid * 8 + g) * MROWS, MROWS)]

    for j in range(S):
        cbuf[pl.ds((S * S + j) * L, L)] = zero

    def compute_group(g, h, in_buf, out_buf, mkbuf, osem, msem):
        @plsc.parallel_loop(0, L)
        def tin_body(l):
            lbase = l * WORDS
            idx0 = lane17 + l
            for rc in range(WORDS // L):
                v = in_buf[pl.ds(lbase + rc * L, L)]
                plsc.store_scatter(xT, [idx0 + rc * L * PAD], v)

        def gram_pass(pairs, rows):
            def body(d, accs):
                d17 = d * PAD
                xs = {j: xT[pl.ds(j * (D * PAD) + d17, L)] for j in rows}
                return tuple(accs[k] + xs[i] * xs[j]
                             for k, (i, j) in enumerate(pairs))
            return lax.fori_loop(0, D, body, tuple(zero for _ in pairs))

        pairs_a = [(i, j) for (i, j) in _PAIRS if i < 3]
        pairs_b = [(i, j) for (i, j) in _PAIRS if i >= 3]
        G_a = gram_pass(pairs_a, range(S))
        G_b = gram_pass(pairs_b, range(3, S))
        gmap = dict(zip(pairs_a, G_a))
        gmap.update(zip(pairs_b, G_b))
        G = tuple(gmap[p] for p in _PAIRS)

        diag = [G[_pair(i, i)] for i in range(S)]
        inv_t = 1.0 / SIM_THRESHOLD
        mb = {}
        mf = {}
        for k, (i, j) in enumerate(_PAIRS):
            lhs = G[k] * inv_t - EPS
            cond = (lhs > 0.0) & (lhs * lhs > diag[i] * diag[j])
            mb[(i, j)] = cond
            mb[(j, i)] = cond
            v = jnp.where(cond, one, zero)
            mf[(i, j)] = v
            mf[(j, i)] = v

        any_offdiag = jnp.any(_tree_or(mb[(i, j)] for (i, j) in _PAIRS
                                       if i != j))

        @pl.when(h > 0)
        def _():
            pltpu.make_async_copy(mkbuf, mask_slice(g), msem).wait()
            pltpu.make_async_copy(out_buf, out_slice(g), osem).wait()

        @pl.when(jnp.logical_not(any_offdiag))
        def _():
            for j in range(S):
                plsc.store_scatter(mkbuf, [lane9 + j], mf[(j, j)])

            @plsc.parallel_loop(0, L)
            def copy_body(l):
                lbase = l * WORDS
                for rc in range(WORDS // L):
                    out_buf[pl.ds(lbase + rc * L, L)] = \
                        in_buf[pl.ds(lbase + rc * L, L)]

            pltpu.async_copy(out_buf, out_slice(g), osem)

        @pl.when(any_offdiag)
        def _():
            nums = [_tree_sum(mf[(i, j)] for j in range(S))
                    for i in range(S)]
            multi = [nums[i] > 1.0 for i in range(S)]
            inv_num = [one / (nums[i] + EPS) for i in range(S)]

            for s in range(S):
                for j in range(S):
                    c = jnp.where(multi[s], mf[(s, j)] * inv_num[s],
                                  one if j == s else zero)
                    cbuf[pl.ds((s * S + j) * L, L)] = c

            min_idx = []
            for i in range(S):
                mi = jnp.zeros((L,), jnp.int32)
                has = jnp.zeros((L,), jnp.bool_)
                for j in range(S):
                    mi = jnp.where(mb[(i, j)] & (~has),
                                   jnp.full((L,), j, jnp.int32), mi)
                    has = has | mb[(i, j)]
                min_idx.append(mi)

            for j in range(S):
                kill = jnp.zeros((L,), jnp.bool_)
                for i in range(S):
                    kill = kill | (mb[(i, j)] & multi[i] & (min_idx[i] != j))
                sm = jnp.where((nums[j] > 0.0) & (~kill), one, zero)
                plsc.store_scatter(mkbuf, [lane9 + j], sm)

            tgt = [jnp.where(multi[s], min_idx[s],
                             jnp.full((L,), s, jnp.int32))
                   for s in range(S)]

            for t0 in range(0, S, 3):
                Cs = []
                for t in range(t0, t0 + 3):
                    w = jnp.full((L,), S, jnp.int32)
                    for s in range(S):
                        w = jnp.where(tgt[s] == t,
                                      jnp.full((L,), s, jnp.int32), w)
                    wrow = w * (S * L)
                    Cs.append([plsc.load_gather(cbuf,
                                                [wrow + (j * L) + lane])
                               for j in range(S)])

                @plsc.parallel_loop(0, D)
                def out_body(d, t0=t0, Cs=Cs):
                    d17 = d * PAD
                    xs = [xT[pl.ds(j * (D * PAD) + d17, L)]
                          for j in range(S)]
                    for k in range(3):
                        t = t0 + k
                        acc = _tree_sum(Cs[k][j] * xs[j] for j in range(S))
                        outT[pl.ds(t * (D * PAD) + d17, L)] = acc

            @plsc.parallel_loop(0, L)
            def tout_body(l):
                lbase = l * WORDS
                idx0 = lane17 + l
                for rc in range(WORDS // L):
                    v = plsc.load_gather(outT, [idx0 + rc * L * PAD])
                    out_buf[pl.ds(lbase + rc * L, L)] = v

            pltpu.async_copy(out_buf, out_slice(g), osem)

        pltpu.async_copy(mkbuf, mask_slice(g), msem)

    pltpu.async_copy(in_slice(0), in0, isem0)

    def pair_body(h, carry):
        g_a = h * 2
        g_b = g_a + 1
        pltpu.async_copy(in_slice(g_b), in1, isem1)
        pltpu.make_async_copy(in_slice(g_a), in0, isem0).wait()
        compute_group(g_a, h, in0, ob0, mk0, osem0, msem0)
        g_n = jnp.minimum(g_a + 2, 7)
        pltpu.async_copy(in_slice(g_n), in0, isem0)
        pltpu.make_async_copy(in_slice(g_b), in1, isem1).wait()
        compute_group(g_b, h, in1, ob1, mk1, osem1, msem1)
        return carry

    lax.fori_loop(0, 4, pair_body, 0)

    pltpu.make_async_copy(in_slice(7), in0, isem0).wait()
    pltpu.make_async_copy(ob0, out_slice(6), osem0).wait()
    pltpu.make_async_copy(ob1, out_slice(7), osem1).wait()
    pltpu.make_async_copy(mk0, mask_slice(6), msem0).wait()
    pltpu.make_async_copy(mk1, mask_slice(7), msem1).wait()


@jax.jit
def _merged(slots_flat):
    B_total = slots_flat.shape[0] // WORDS
    run = pl.kernel(
        _sc_body,
        out_type=(
            jax.ShapeDtypeStruct((B_total * WORDS,), jnp.float32),
            jax.ShapeDtypeStruct((B_total * S,), jnp.float32),
        ),
        mesh=plsc.VectorSubcoreMesh(core_axis_name="c",
                                    subcore_axis_name="s"),
        compiler_params=pltpu.CompilerParams(needs_layout_passes=False),
        scratch_types=[
            pltpu.VMEM((GROUP_WORDS,), jnp.float32),
            pltpu.VMEM((GROUP_WORDS,), jnp.float32),
            pltpu.VMEM((GROUP_WORDS,), jnp.float32),
            pltpu.VMEM((GROUP_WORDS,), jnp.float32),
            pltpu.VMEM((TWORDS,), jnp.float32),
            pltpu.VMEM((TWORDS,), jnp.float32),
            pltpu.VMEM((CROWS * L,), jnp.float32),
            pltpu.VMEM((MROWS,), jnp.float32),
            pltpu.VMEM((MROWS,), jnp.float32),
            pltpu.SemaphoreType.DMA,
            pltpu.SemaphoreType.DMA,
            pltpu.SemaphoreType.DMA,
            pltpu.SemaphoreType.DMA,
            pltpu.SemaphoreType.DMA,
            pltpu.SemaphoreType.DMA,
        ],
    )
    return run(slots_flat)


def kernel(slots):
    B, s, d = slots.shape
    final_flat, mask_flat = _merged(slots.reshape(-1))
    return final_flat.reshape(B, s, d), mask_flat.reshape(B, s)


# speedup vs baseline: 1.0133x; 1.0002x over previous
import jax
import jax.numpy as jnp
from jax import lax
from jax.experimental import pallas as pl
from jax.experimental.pallas import tpu as pltpu
from jax.experimental.pallas import tpu_sc as plsc

SIM_THRESHOLD = 0.9
EPS = 1e-08

S = 9
D = 64
L = 16
NC = 2
NS = 16
NW = NC * NS
WORDS = S * D
GROUP_WORDS = L * WORDS
PAD = 17
TWORDS = WORDS * PAD
CROWS = S * S + S
MROWS = L * S

_PAIRS = [(i, j) for i in range(S) for j in range(i, S)]
_PIDX = {(i, j): k for k, (i, j) in enumerate(_PAIRS)}


def _pair(i, j):
    return _PIDX[(i, j)] if i <= j else _PIDX[(j, i)]


def _tree_or(vals):
    vals = list(vals)
    while len(vals) > 1:
        nxt = [vals[k] | vals[k + 1] for k in range(0, len(vals) - 1, 2)]
        if len(vals) % 2:
            nxt.append(vals[-1])
        vals = nxt
    return vals[0]


def _tree_sum(vals):
    vals = list(vals)
    while len(vals) > 1:
        nxt = [vals[k] + vals[k + 1] for k in range(0, len(vals) - 1, 2)]
        if len(vals) % 2:
            nxt.append(vals[-1])
        vals = nxt
    return vals[0]


def _sc_body(slots_hbm, final_hbm, mask_hbm,
             in0, in1, ob0, ob1, xT, outT, cbuf, mk0, mk1,
             isem0, isem1, osem0, osem1, msem0, msem1):
    wid = lax.axis_index("c") * NS + lax.axis_index("s")
    lane = lax.iota(jnp.int32, L)
    lane9 = lane * S
    lane17 = lane * PAD
    zero = jnp.zeros((L,), jnp.float32)
    one = jnp.ones((L,), jnp.float32)

    def in_slice(g):
        return slots_hbm.at[pl.ds((wid * 8 + g) * GROUP_WORDS, GROUP_WORDS)]

    def out_slice(g):
        return final_hbm.at[pl.ds((wid * 8 + g) * GROUP_WORDS, GROUP_WORDS)]

    def mask_slice(g):
        return mask_hbm.at[pl.ds((wid * 8 + g) * MROWS, MROWS)]

    for j in range(S):
        cbuf[pl.ds((S * S + j) * L, L)] = zero

    def compute_group(g, h, in_buf, out_buf, mkbuf, osem, msem):
        @plsc.parallel_loop(0, L)
        def tin_body(l):
            lbase = l * WORDS
            idx0 = lane17 + l
            for rc in range(WORDS // L):
                v = in_buf[pl.ds(lbase + rc * L, L)]
                plsc.store_scatter(xT, [idx0 + rc * L * PAD], v)

        def gram_pass(pairs, rows):
            def body(d, accs):
                d17 = d * PAD
                xs = {j: xT[pl.ds(j * (D * PAD) + d17, L)] for j in rows}
                return tuple(accs[k] + xs[i] * xs[j]
                             for k, (i, j) in enumerate(pairs))
            return lax.fori_loop(0, D, body, tuple(zero for _ in pairs))

        pairs_a = [(i, j) for (i, j) in _PAIRS if i < 3]
        pairs_b = [(i, j) for (i, j) in _PAIRS if i >= 3]
        G_a = gram_pass(pairs_a, range(S))
        G_b = gram_pass(pairs_b, range(3, S))
        gmap = dict(zip(pairs_a, G_a))
        gmap.update(zip(pairs_b, G_b))
        G = tuple(gmap[p] for p in _PAIRS)

        diag = [G[_pair(i, i)] for i in range(S)]
        inv_t = 1.0 / SIM_THRESHOLD
        mb = {}
        mf = {}
        for k, (i, j) in enumerate(_PAIRS):
            lhs = G[k] * inv_t - EPS
            cond = (lhs > 0.0) & (lhs * lhs > diag[i] * diag[j])
            mb[(i, j)] = cond
            mb[(j, i)] = cond
            v = jnp.where(cond, one, zero)
            mf[(i, j)] = v
            mf[(j, i)] = v

        any_offdiag = jnp.any(_tree_or(mb[(i, j)] for (i, j) in _PAIRS
                                       if i != j))

        @pl.when(h > 0)
        def _():
            pltpu.make_async_copy(mkbuf, mask_slice(g), msem).wait()
            pltpu.make_async_copy(out_buf, out_slice(g), osem).wait()

        @pl.when(jnp.logical_not(any_offdiag))
        def _():
            for j in range(S):
                plsc.store_scatter(mkbuf, [lane9 + j], mf[(j, j)])

            @plsc.parallel_loop(0, L)
            def copy_body(l):
                lbase = l * WORDS
                for rc in range(WORDS // L):
                    out_buf[pl.ds(lbase + rc * L, L)] = \
                        in_buf[pl.ds(lbase + rc * L, L)]

            pltpu.async_copy(out_buf, out_slice(g), osem)

        @pl.when(any_offdiag)
        def _():
            nums = [_tree_sum(mf[(i, j)] for j in range(S))
                    for i in range(S)]
            multi = [nums[i] > 1.0 for i in range(S)]
            inv_num = [one / (nums[i] + EPS) for i in range(S)]

            for s in range(S):
                for j in range(S):
                    c = jnp.where(multi[s], mf[(s, j)] * inv_num[s],
                                  one if j == s else zero)
                    cbuf[pl.ds((s * S + j) * L, L)] = c

            min_idx = []
            for i in range(S):
                mi = jnp.zeros((L,), jnp.int32)
                has = jnp.zeros((L,), jnp.bool_)
                for j in range(S):
                    mi = jnp.where(mb[(i, j)] & (~has),
                                   jnp.full((L,), j, jnp.int32), mi)
                    has = has | mb[(i, j)]
                min_idx.append(mi)

            for j in range(S):
                kill = jnp.zeros((L,), jnp.bool_)
                for i in range(S):
                    kill = kill | (mb[(i, j)] & multi[i] & (min_idx[i] != j))
                sm = jnp.where((nums[j] > 0.0) & (~kill), one, zero)
                plsc.store_scatter(mkbuf, [lane9 + j], sm)

            tgt = [jnp.where(multi[s], min_idx[s],
                             jnp.full((L,), s, jnp.int32))
                   for s in range(S)]

            for t0 in range(0, S, 3):
                Cs = []
                for t in range(t0, t0 + 3):
                    w = jnp.full((L,), S, jnp.int32)
                    for s in range(S):
                        w = jnp.where(tgt[s] == t,
                                      jnp.full((L,), s, jnp.int32), w)
                    wrow = w * (S * L)
                    Cs.append([plsc.load_gather(cbuf,
                                                [wrow + (j * L) + lane])
                               for j in range(S)])

                @plsc.parallel_loop(0, D)
                def out_body(d, t0=t0, Cs=Cs):
                    d17 = d * PAD
                    xs = [xT[pl.ds(j * (D * PAD) + d17, L)]
                          for j in range(S)]
                    for k in range(3):
                        t = t0 + k
                        acc = _tree_sum(Cs[k][j] * xs[j] for j in range(S))
                        outT[pl.ds(t * (D * PAD) + d17, L)] = acc

            @plsc.parallel_loop(0, L)
            def tout_body(l):
                lbase = l * WORDS
                idx0 = lane17 + l
                for rc in range(WORDS // L):
                    v = plsc.load_gather(outT, [idx0 + rc * L * PAD])
                    out_buf[pl.ds(lbase + rc * L, L)] = v

            pltpu.async_copy(out_buf, out_slice(g), osem)

        pltpu.async_copy(mkbuf, mask_slice(g), msem)

    pltpu.async_copy(in_slice(0), in0, isem0)

    def pair_body(h, carry):
        g_a = h * 2
        g_b = g_a + 1
        pltpu.async_copy(in_slice(g_b), in1, isem1)
        pltpu.make_async_copy(in_slice(g_a), in0, isem0).wait()
        compute_group(g_a, h, in0, ob0, mk0, osem0, msem0)
        g_n = jnp.minimum(g_a + 2, 7)
        pltpu.async_copy(in_slice(g_n), in0, isem0)
        pltpu.make_async_copy(in_slice(g_b), in1, isem1).wait()
        compute_group(g_b, h, in1, ob1, mk1, osem1, msem1)
        return carry

    lax.fori_loop(0, 4, pair_body, 0)

    pltpu.make_async_copy(in_slice(7), in0, isem0).wait()
    pltpu.make_async_copy(ob0, out_slice(6), osem0).wait()
    pltpu.make_async_copy(ob1, out_slice(7), osem1).wait()
    pltpu.make_async_copy(mk0, mask_slice(6), msem0).wait()
    pltpu.make_async_copy(mk1, mask_slice(7), msem1).wait()


@jax.jit
def _merged(slots_flat):
    B_total = slots_flat.shape[0] // WORDS
    run = pl.kernel(
        _sc_body,
        out_type=(
            jax.ShapeDtypeStruct((B_total * WORDS,), jnp.float32),
            jax.ShapeDtypeStruct((B_total * S,), jnp.float32),
        ),
        mesh=plsc.VectorSubcoreMesh(core_axis_name="c",
                                    subcore_axis_name="s"),
        compiler_params=pltpu.CompilerParams(needs_layout_passes=False),
        scratch_types=[
            pltpu.VMEM((GROUP_WORDS,), jnp.float32),
            pltpu.VMEM((GROUP_WORDS,), jnp.float32),
            pltpu.VMEM((GROUP_WORDS,), jnp.float32),
            pltpu.VMEM((GROUP_WORDS,), jnp.float32),
            pltpu.VMEM((TWORDS,), jnp.float32),
            pltpu.VMEM((TWORDS,), jnp.float32),
            pltpu.VMEM((CROWS * L,), jnp.float32),
            pltpu.VMEM((MROWS,), jnp.float32),
            pltpu.VMEM((MROWS,), jnp.float32),
            pltpu.SemaphoreType.DMA,
            pltpu.SemaphoreType.DMA,
            pltpu.SemaphoreType.DMA,
            pltpu.SemaphoreType.DMA,
            pltpu.SemaphoreType.DMA,
            pltpu.SemaphoreType.DMA,
        ],
    )
    return run(slots_flat)


def kernel(slots):
    B, s, d = slots.shape
    final_flat, mask_flat = _merged(slots.reshape(-1))
    return final_flat.reshape(B, s, d), mask_flat.reshape(B, s)
